# baseline (device time: 52749 ns/iter reference)
import os

import jax
import jax.numpy as jnp
from jax import lax
from jax.experimental import pallas as pl
from jax.experimental.pallas import tpu as pltpu

_NOCOMM = bool(os.environ.get("KDBG_NOCOMM"))
_NOGEMM = bool(os.environ.get("KDBG_NOGEMM"))

N_DEV = 4
H = 2
N_XSTRIPS = 8
ORDER = [2, 1, 3, 0]
SB = {2: 0, 1: 1, 3: 2}
EP_ORDER = [0, 2, 3, 1]


def kernel(x, w_mat):
    m_per, k = x.shape
    _, n = w_mat.shape
    n_per = n // N_DEV
    hm = m_per // H
    sm = m_per // N_XSTRIPS

    def body(x_hbm, w_hbm, out_hbm,
             xstrip, xbf, wbuf, sendbuf, recvbuf, q8buf, q8recv, outv,
             amax_send, amax_recv,
             x_sems, w_sems, out_sems, send_sems, recv_sems,
             q8_send_sems, q8_recv_sems, amax_send_sems, amax_recv_sems):
        my_i = lax.axis_index("i")

        if not _NOCOMM:
            barrier = pltpu.get_barrier_semaphore()
            for p in range(N_DEV):
                @pl.when(my_i != p)
                def _():
                    pl.semaphore_signal(
                        barrier, inc=1,
                        device_id=(p,), device_id_type=pl.DeviceIdType.MESH,
                    )
            pl.semaphore_wait(barrier, N_DEV - 1)

        targets = [(my_i + d) % N_DEV for d in ORDER]

        def x_dma(s, slot):
            return pltpu.make_async_copy(
                x_hbm.at[pl.ds(s * sm, sm), :], xstrip.at[slot],
                x_sems.at[slot],
            )

        def w_dma(idx, slot):
            return pltpu.make_async_copy(
                w_hbm.at[:, pl.ds(targets[idx] * n_per, n_per)],
                wbuf.at[slot], w_sems.at[slot],
            )

        def cvt_strips(lo, hi):
            for s in range(lo, hi):
                x_dma(s, s % 2).wait()
                xbf[pl.ds(s * sm, sm), :] = xstrip[s % 2].astype(jnp.bfloat16)
                if s + 2 < N_XSTRIPS:
                    x_dma(s + 2, s % 2).start()

        with jax.named_scope("xload"):
            w_dma(0, 0).start()
            x_dma(0, 0).start()
            x_dma(1, 1).start()
            cvt_strips(0, N_XSTRIPS // 2)

        amax_val = jnp.float32(0)
        for idx, d in enumerate(ORDER):
            scope = jax.named_scope(f"chunk{idx}")
            scope.__enter__()
            slot = idx % 2
            if idx + 1 < N_DEV:
                w_dma(idx + 1, (idx + 1) % 2).start()
            w_dma(idx, slot).wait()
            for h in range(H):
                if _NOGEMM:
                    yh = xbf[pl.ds(h * hm, hm), :n_per].astype(jnp.float32)
                else:
                    yh = jnp.dot(
                        xbf[pl.ds(h * hm, hm), :],
                        wbuf[slot].astype(jnp.bfloat16),
                        preferred_element_type=jnp.float32,
                    )
                amax_val = jnp.maximum(amax_val, jnp.max(yh))
                yb = yh.astype(jnp.bfloat16)

                if d == 0:
                    for o in range(N_DEV):
                        @pl.when(my_i == o)
                        def _():
                            recvbuf[o, h * hm:(h + 1) * hm, :] = yb
                else:
                    sb = SB[d]
                    sendbuf[sb, h * hm:(h + 1) * hm, :] = yb
                    if h == 0 and not _NOCOMM:
                        rdma = pltpu.make_async_remote_copy(
                            src_ref=sendbuf.at[sb, pl.ds(0, hm), :],
                            dst_ref=recvbuf.at[my_i, pl.ds(0, hm), :],
                            send_sem=send_sems.at[sb],
                            recv_sem=recv_sems.at[my_i],
                            device_id=(targets[idx],),
                            device_id_type=pl.DeviceIdType.MESH,
                        )
                        rdma.start()
                if idx == 0 and h == 0:
                    cvt_strips(N_XSTRIPS // 2, N_XSTRIPS)
            scope.__exit__(None, None, None)

        _amax_scope = jax.named_scope("amax_exchange")
        _amax_scope.__enter__()
        amax_send[:, :] = jnp.full((8, 128), amax_val, jnp.float32)
        for p in range(N_DEV if not _NOCOMM else 0):
            @pl.when(my_i == p)
            def _():
                amax_recv[p, :, :] = jnp.full((8, 128), amax_val, jnp.float32)

            @pl.when(my_i != p)
            def _():
                rdma = pltpu.make_async_remote_copy(
                    src_ref=amax_send,
                    dst_ref=amax_recv.at[my_i],
                    send_sem=amax_send_sems.at[p],
                    recv_sem=amax_recv_sems.at[my_i],
                    device_id=(p,),
                    device_id_type=pl.DeviceIdType.MESH,
                )
                rdma.start()
        for p in range(N_DEV if not _NOCOMM else 0):
            @pl.when(my_i != p)
            def _():
                am = pltpu.make_async_remote_copy(
                    src_ref=amax_send, dst_ref=amax_recv.at[p],
                    send_sem=amax_send_sems.at[p],
                    recv_sem=amax_recv_sems.at[p],
                    device_id=(p,), device_id_type=pl.DeviceIdType.MESH,
                )
                am.wait_send()
                am.wait_recv()

        if _NOCOMM:
            g_amax = (amax_val + amax_send[0, 0]) * 0.5
        else:
            g_amax = amax_recv[0, 0, 0]
            for p in range(1, N_DEV):
                g_amax = jnp.maximum(g_amax, amax_recv[p, 0, 0])
        scale = g_amax / 448.0
        inv_scale = 448.0 / g_amax

        _amax_scope.__exit__(None, None, None)

        def quant(v32):
            return jnp.minimum(v32 * inv_scale, 448.0).astype(
                jnp.float8_e4m3fn)

        _p2 = jax.named_scope("phase2_sends")
        _p2.__enter__()
        if not _NOCOMM:
            for idx, d in enumerate(ORDER[:3]):
                sb = SB[d]
                v = jnp.maximum(
                    sendbuf[sb, hm:2 * hm, :].astype(jnp.float32), 0.0)
                q8buf[sb, :, :] = quant(v)
                rdma = pltpu.make_async_remote_copy(
                    src_ref=q8buf.at[sb],
                    dst_ref=q8recv.at[my_i],
                    send_sem=q8_send_sems.at[sb],
                    recv_sem=q8_recv_sems.at[my_i],
                    device_id=(targets[idx],),
                    device_id_type=pl.DeviceIdType.MESH,
                )
                rdma.start()

        _p2.__exit__(None, None, None)

        def out_half_dma(oslot, o, h):
            return pltpu.make_async_copy(
                outv.at[oslot],
                out_hbm.at[pl.ds(o * m_per + h * hm, hm), :],
                out_sems.at[oslot],
            )

        _pa = jax.named_scope("passA")
        _pa.__enter__()
        jobs_a = []
        for c, d in enumerate(EP_ORDER):
            jobs_a.append((d, 0))
        jobs_a.insert(1, (0, 1))
        nslot = 0
        inflight = []
        for d, h in jobs_a:
            o = (my_i + d) % N_DEV
            oslot = nslot % 2
            nslot += 1
            if len(inflight) >= 2:
                ps, po, ph = inflight.pop(0)
                out_half_dma(ps, po, ph).wait()
            if d != 0 and h == 0 and not _NOCOMM:
                rcv = pltpu.make_async_remote_copy(
                    src_ref=sendbuf.at[0, pl.ds(0, hm), :],
                    dst_ref=recvbuf.at[o, pl.ds(0, hm), :],
                    send_sem=send_sems.at[0],
                    recv_sem=recv_sems.at[o],
                    device_id=(o,), device_id_type=pl.DeviceIdType.MESH,
                )
                rcv.wait_recv()
            v = jnp.maximum(
                recvbuf[o, h * hm:(h + 1) * hm, :].astype(jnp.float32), 0.0)
            outv[oslot, :, :] = quant(v).astype(jnp.float32) * scale
            out_half_dma(oslot, o, h).start()
            inflight.append((oslot, o, h))

        _pa.__exit__(None, None, None)
        _pb = jax.named_scope("passB")
        _pb.__enter__()
        for c, d in enumerate(EP_ORDER[1:]):
            o = (my_i + d) % N_DEV
            oslot = nslot % 2
            nslot += 1
            if len(inflight) >= 2:
                ps, po, ph = inflight.pop(0)
                out_half_dma(ps, po, ph).wait()
            if not _NOCOMM:
                rcv = pltpu.make_async_remote_copy(
                    src_ref=q8buf.at[0],
                    dst_ref=q8recv.at[o],
                    send_sem=q8_send_sems.at[0],
                    recv_sem=q8_recv_sems.at[o],
                    device_id=(o,), device_id_type=pl.DeviceIdType.MESH,
                )
                rcv.wait_recv()
            outv[oslot, :, :] = q8recv[o].astype(jnp.float32) * scale
            out_half_dma(oslot, o, 1).start()
            inflight.append((oslot, o, 1))

        _pb.__exit__(None, None, None)
        for ps, po, ph in inflight:
            out_half_dma(ps, po, ph).wait()

        for sb in range(N_DEV - 1 if not _NOCOMM else 0):
            snd = pltpu.make_async_remote_copy(
                src_ref=sendbuf.at[sb, pl.ds(0, hm), :],
                dst_ref=recvbuf.at[0, pl.ds(0, hm), :],
                send_sem=send_sems.at[sb],
                recv_sem=recv_sems.at[0],
                device_id=(0,), device_id_type=pl.DeviceIdType.MESH,
            )
            snd.wait_send()
            q8s = pltpu.make_async_remote_copy(
                src_ref=q8buf.at[sb],
                dst_ref=q8recv.at[0],
                send_sem=q8_send_sems.at[sb],
                recv_sem=q8_recv_sems.at[0],
                device_id=(0,), device_id_type=pl.DeviceIdType.MESH,
            )
            q8s.wait_send()

    return pl.pallas_call(
        body,
        out_shape=jax.ShapeDtypeStruct((N_DEV * m_per, n_per), jnp.float32),
        in_specs=[
            pl.BlockSpec(memory_space=pl.ANY),
            pl.BlockSpec(memory_space=pl.ANY),
        ],
        out_specs=pl.BlockSpec(memory_space=pl.ANY),
        scratch_shapes=[
            pltpu.VMEM((2, sm, k), jnp.float32),
            pltpu.VMEM((m_per, k), jnp.bfloat16),
            pltpu.VMEM((2, k, n_per), jnp.float32),
            pltpu.VMEM((3, m_per, n_per), jnp.bfloat16),
            pltpu.VMEM((N_DEV, m_per, n_per), jnp.bfloat16),
            pltpu.VMEM((3, hm, n_per), jnp.float8_e4m3fn),
            pltpu.VMEM((N_DEV, hm, n_per), jnp.float8_e4m3fn),
            pltpu.VMEM((2, hm, n_per), jnp.float32),
            pltpu.VMEM((8, 128), jnp.float32),
            pltpu.VMEM((N_DEV, 8, 128), jnp.float32),
            pltpu.SemaphoreType.DMA((2,)),
            pltpu.SemaphoreType.DMA((2,)),
            pltpu.SemaphoreType.DMA((2,)),
            pltpu.SemaphoreType.DMA((3,)),
            pltpu.SemaphoreType.DMA((N_DEV,)),
            pltpu.SemaphoreType.DMA((3,)),
            pltpu.SemaphoreType.DMA((N_DEV,)),
            pltpu.SemaphoreType.DMA((N_DEV,)),
            pltpu.SemaphoreType.DMA((N_DEV,)),
        ],
        compiler_params=pltpu.CompilerParams(
            collective_id=None if _NOCOMM else 0,
            vmem_limit_bytes=56 * 1024 * 1024,
        ),
    )(x, w_mat)


# device time: 50842 ns/iter; 1.0375x vs baseline; 1.0375x over previous
import os

import jax
import jax.numpy as jnp
from jax import lax
from jax.experimental import pallas as pl
from jax.experimental.pallas import tpu as pltpu

_NOCOMM = bool(os.environ.get("KDBG_NOCOMM"))
_NOGEMM = bool(os.environ.get("KDBG_NOGEMM"))

N_DEV = 4
H = 2
N_XSTRIPS = 8
ORDER = [2, 1, 3, 0]
SB = {2: 0, 1: 1, 3: 2}
EP_ORDER = [0, 2, 3, 1]


def kernel(x, w_mat):
    m_per, k = x.shape
    _, n = w_mat.shape
    n_per = n // N_DEV
    hm = m_per // H
    sm = m_per // N_XSTRIPS

    def body(x_hbm, w_hbm, out_hbm,
             xstrip, xbf, wbuf, sendbuf, recvbuf, outv,
             amax_send, amax_recv,
             x_sems, w_sems, out_sems, send_sems, recv_sems,
             amax_send_sems, amax_recv_sems):
        my_i = lax.axis_index("i")

        if not _NOCOMM:
            barrier = pltpu.get_barrier_semaphore()
            for p in range(N_DEV):
                @pl.when(my_i != p)
                def _():
                    pl.semaphore_signal(
                        barrier, inc=1,
                        device_id=(p,), device_id_type=pl.DeviceIdType.MESH,
                    )
            pl.semaphore_wait(barrier, N_DEV - 1)

        targets = [(my_i + d) % N_DEV for d in ORDER]

        def x_dma(s, slot):
            return pltpu.make_async_copy(
                x_hbm.at[pl.ds(s * sm, sm), :], xstrip.at[slot],
                x_sems.at[slot],
            )

        def w_dma(idx, slot):
            return pltpu.make_async_copy(
                w_hbm.at[:, pl.ds(targets[idx] * n_per, n_per)],
                wbuf.at[slot], w_sems.at[slot],
            )

        def cvt_strips(lo, hi):
            for s in range(lo, hi):
                x_dma(s, s % 2).wait()
                xbf[pl.ds(s * sm, sm), :] = xstrip[s % 2].astype(jnp.bfloat16)
                if s + 2 < N_XSTRIPS:
                    x_dma(s + 2, s % 2).start()

        w_dma(0, 0).start()
        x_dma(0, 0).start()
        x_dma(1, 1).start()
        cvt_strips(0, N_XSTRIPS // 2)

        amax_val = jnp.float32(0)
        for idx, d in enumerate(ORDER):
            slot = idx % 2
            if idx + 1 < N_DEV:
                w_dma(idx + 1, (idx + 1) % 2).start()
            w_dma(idx, slot).wait()
            for h in range(H):
                if _NOGEMM:
                    yh = xbf[pl.ds(h * hm, hm), :n_per].astype(jnp.float32)
                else:
                    yh = jnp.dot(
                        xbf[pl.ds(h * hm, hm), :],
                        wbuf[slot].astype(jnp.bfloat16),
                        preferred_element_type=jnp.float32,
                    )
                amax_val = jnp.maximum(amax_val, jnp.max(yh))
                yb = yh.astype(jnp.bfloat16)

                if d == 0:
                    for o in range(N_DEV):
                        @pl.when(my_i == o)
                        def _():
                            recvbuf[o, h * hm:(h + 1) * hm, :] = yb
                else:
                    sb = SB[d]
                    sendbuf[sb, h * hm:(h + 1) * hm, :] = yb
                    if not _NOCOMM:
                        rdma = pltpu.make_async_remote_copy(
                            src_ref=sendbuf.at[sb, pl.ds(h * hm, hm), :],
                            dst_ref=recvbuf.at[my_i, pl.ds(h * hm, hm), :],
                            send_sem=send_sems.at[sb * H + h],
                            recv_sem=recv_sems.at[my_i * H + h],
                            device_id=(targets[idx],),
                            device_id_type=pl.DeviceIdType.MESH,
                        )
                        rdma.start()
                if idx == 0 and h == 0:
                    cvt_strips(N_XSTRIPS // 2, N_XSTRIPS)

        amax_send[:, :] = jnp.full((8, 128), amax_val, jnp.float32)
        for p in range(N_DEV if not _NOCOMM else 0):
            @pl.when(my_i == p)
            def _():
                amax_recv[p, :, :] = jnp.full((8, 128), amax_val, jnp.float32)

            @pl.when(my_i != p)
            def _():
                rdma = pltpu.make_async_remote_copy(
                    src_ref=amax_send,
                    dst_ref=amax_recv.at[my_i],
                    send_sem=amax_send_sems.at[p],
                    recv_sem=amax_recv_sems.at[my_i],
                    device_id=(p,),
                    device_id_type=pl.DeviceIdType.MESH,
                )
                rdma.start()
        for p in range(N_DEV if not _NOCOMM else 0):
            @pl.when(my_i != p)
            def _():
                am = pltpu.make_async_remote_copy(
                    src_ref=amax_send, dst_ref=amax_recv.at[p],
                    send_sem=amax_send_sems.at[p],
                    recv_sem=amax_recv_sems.at[p],
                    device_id=(p,), device_id_type=pl.DeviceIdType.MESH,
                )
                am.wait_send()
                am.wait_recv()

        if _NOCOMM:
            g_amax = amax_val + amax_send[0, 0]
        else:
            g_amax = amax_recv[0, 0, 0]
            for p in range(1, N_DEV):
                g_amax = jnp.maximum(g_amax, amax_recv[p, 0, 0])
        scale = g_amax / 448.0
        inv_scale = 448.0 / g_amax

        for c, d in enumerate(EP_ORDER):
            o = (my_i + d) % N_DEV
            oslot = c % 2
            if c >= 2:
                o_prev = (my_i + EP_ORDER[c - 2]) % N_DEV
                pltpu.make_async_copy(
                    outv.at[oslot],
                    out_hbm.at[pl.ds(o_prev * m_per, m_per), :],
                    out_sems.at[oslot],
                ).wait()
            if d != 0 and not _NOCOMM:
                for h in range(H):
                    rcv = pltpu.make_async_remote_copy(
                        src_ref=sendbuf.at[0, pl.ds(h * hm, hm), :],
                        dst_ref=recvbuf.at[o, pl.ds(h * hm, hm), :],
                        send_sem=send_sems.at[0],
                        recv_sem=recv_sems.at[o * H + h],
                        device_id=(o,), device_id_type=pl.DeviceIdType.MESH,
                    )
                    rcv.wait_recv()
            v = jnp.maximum(recvbuf[o].astype(jnp.float32), 0.0)
            q = jnp.minimum(v * inv_scale, 448.0).astype(jnp.float8_e4m3fn)
            outv[oslot, :, :] = q.astype(jnp.float32) * scale
            pltpu.make_async_copy(
                outv.at[oslot],
                out_hbm.at[pl.ds(o * m_per, m_per), :],
                out_sems.at[oslot],
            ).start()

        for c in (2, 3):
            o = (my_i + EP_ORDER[c]) % N_DEV
            pltpu.make_async_copy(
                outv.at[c % 2],
                out_hbm.at[pl.ds(o * m_per, m_per), :],
                out_sems.at[c % 2],
            ).wait()
        for sb in range(N_DEV - 1 if not _NOCOMM else 0):
            for h in range(H):
                snd = pltpu.make_async_remote_copy(
                    src_ref=sendbuf.at[sb, pl.ds(h * hm, hm), :],
                    dst_ref=recvbuf.at[0, pl.ds(h * hm, hm), :],
                    send_sem=send_sems.at[sb * H + h],
                    recv_sem=recv_sems.at[0],
                    device_id=(0,), device_id_type=pl.DeviceIdType.MESH,
                )
                snd.wait_send()

    return pl.pallas_call(
        body,
        out_shape=jax.ShapeDtypeStruct((N_DEV * m_per, n_per), jnp.float32),
        in_specs=[
            pl.BlockSpec(memory_space=pl.ANY),
            pl.BlockSpec(memory_space=pl.ANY),
        ],
        out_specs=pl.BlockSpec(memory_space=pl.ANY),
        scratch_shapes=[
            pltpu.VMEM((2, sm, k), jnp.float32),
            pltpu.VMEM((m_per, k), jnp.bfloat16),
            pltpu.VMEM((2, k, n_per), jnp.float32),
            pltpu.VMEM((3, m_per, n_per), jnp.bfloat16),
            pltpu.VMEM((N_DEV, m_per, n_per), jnp.bfloat16),
            pltpu.VMEM((2, m_per, n_per), jnp.float32),
            pltpu.VMEM((8, 128), jnp.float32),
            pltpu.VMEM((N_DEV, 8, 128), jnp.float32),
            pltpu.SemaphoreType.DMA((2,)),
            pltpu.SemaphoreType.DMA((2,)),
            pltpu.SemaphoreType.DMA((2,)),
            pltpu.SemaphoreType.DMA((3 * H,)),
            pltpu.SemaphoreType.DMA((N_DEV * H,)),
            pltpu.SemaphoreType.DMA((N_DEV,)),
            pltpu.SemaphoreType.DMA((N_DEV,)),
        ],
        compiler_params=pltpu.CompilerParams(
            collective_id=None if _NOCOMM else 0,
            vmem_limit_bytes=56 * 1024 * 1024,
        ),
    )(x, w_mat)


# device time: 50579 ns/iter; 1.0429x vs baseline; 1.0052x over previous
import os

import jax
import jax.numpy as jnp
from jax import lax
from jax.experimental import pallas as pl
from jax.experimental.pallas import tpu as pltpu

_NOCOMM = bool(os.environ.get("KDBG_NOCOMM"))
_NOGEMM = bool(os.environ.get("KDBG_NOGEMM"))

N_DEV = 4
H = 2
N_XSTRIPS = 8
ORDER = [2, 1, 3, 0]
SB = {2: 0, 1: 1, 3: 2}
EP_ORDER = [0, 2, 3, 1]


def kernel(x, w_mat):
    m_per, k = x.shape
    _, n = w_mat.shape
    n_per = n // N_DEV
    hm = m_per // H
    sm = m_per // N_XSTRIPS

    def body(x_hbm, w_hbm, out_hbm,
             xstrip, xbf, wbuf, sendbuf, recvbuf, outv,
             amax_send, amax_recv,
             x_sems, w_sems, out_sems, send_sems, recv_sems,
             amax_send_sems, amax_recv_sems):
        my_i = lax.axis_index("i")

        if not _NOCOMM:
            barrier = pltpu.get_barrier_semaphore()
            for p in range(N_DEV):
                @pl.when(my_i != p)
                def _():
                    pl.semaphore_signal(
                        barrier, inc=1,
                        device_id=(p,), device_id_type=pl.DeviceIdType.MESH,
                    )
            pl.semaphore_wait(barrier, N_DEV - 1)

        targets = [(my_i + d) % N_DEV for d in ORDER]

        def x_dma(s, slot):
            return pltpu.make_async_copy(
                x_hbm.at[pl.ds(s * sm, sm), :], xstrip.at[slot],
                x_sems.at[slot],
            )

        def w_dma(idx, slot):
            return pltpu.make_async_copy(
                w_hbm.at[:, pl.ds(targets[idx] * n_per, n_per)],
                wbuf.at[slot], w_sems.at[slot],
            )

        def cvt_strips(lo, hi):
            for s in range(lo, hi):
                x_dma(s, s % 2).wait()
                xbf[pl.ds(s * sm, sm), :] = xstrip[s % 2].astype(jnp.bfloat16)
                if s + 2 < N_XSTRIPS:
                    x_dma(s + 2, s % 2).start()

        w_dma(0, 0).start()
        x_dma(0, 0).start()
        x_dma(1, 1).start()
        cvt_strips(0, N_XSTRIPS // 2)

        amax_acc = jnp.zeros((hm, n_per), jnp.float32)
        for idx, d in enumerate(ORDER):
            slot = idx % 2
            if idx + 1 < N_DEV:
                w_dma(idx + 1, (idx + 1) % 2).start()
            w_dma(idx, slot).wait()
            for h in range(H):
                if _NOGEMM:
                    yh = xbf[pl.ds(h * hm, hm), :n_per].astype(jnp.float32)
                else:
                    yh = jnp.dot(
                        xbf[pl.ds(h * hm, hm), :],
                        wbuf[slot].astype(jnp.bfloat16),
                        preferred_element_type=jnp.float32,
                    )
                amax_acc = jnp.maximum(amax_acc, yh)
                yb = yh.astype(jnp.bfloat16)

                if d == 0:
                    for o in range(N_DEV):
                        @pl.when(my_i == o)
                        def _():
                            recvbuf[o, h * hm:(h + 1) * hm, :] = yb
                else:
                    sb = SB[d]
                    sendbuf[sb, h * hm:(h + 1) * hm, :] = yb
                    if not _NOCOMM:
                        rdma = pltpu.make_async_remote_copy(
                            src_ref=sendbuf.at[sb, pl.ds(h * hm, hm), :],
                            dst_ref=recvbuf.at[my_i, pl.ds(h * hm, hm), :],
                            send_sem=send_sems.at[sb * H + h],
                            recv_sem=recv_sems.at[my_i * H + h],
                            device_id=(targets[idx],),
                            device_id_type=pl.DeviceIdType.MESH,
                        )
                        rdma.start()
                if idx == 0 and h == 0:
                    cvt_strips(N_XSTRIPS // 2, N_XSTRIPS)

        amax_val = jnp.max(amax_acc)
        amax_send[:, :] = jnp.full((8, 128), amax_val, jnp.float32)
        for p in range(N_DEV if not _NOCOMM else 0):
            @pl.when(my_i == p)
            def _():
                amax_recv[p, :, :] = jnp.full((8, 128), amax_val, jnp.float32)

            @pl.when(my_i != p)
            def _():
                rdma = pltpu.make_async_remote_copy(
                    src_ref=amax_send,
                    dst_ref=amax_recv.at[my_i],
                    send_sem=amax_send_sems.at[p],
                    recv_sem=amax_recv_sems.at[my_i],
                    device_id=(p,),
                    device_id_type=pl.DeviceIdType.MESH,
                )
                rdma.start()
        for p in range(N_DEV if not _NOCOMM else 0):
            @pl.when(my_i != p)
            def _():
                am = pltpu.make_async_remote_copy(
                    src_ref=amax_send, dst_ref=amax_recv.at[p],
                    send_sem=amax_send_sems.at[p],
                    recv_sem=amax_recv_sems.at[p],
                    device_id=(p,), device_id_type=pl.DeviceIdType.MESH,
                )
                am.wait_send()
                am.wait_recv()

        if _NOCOMM:
            g_amax = amax_val + amax_send[0, 0]
        else:
            g_amax = amax_recv[0, 0, 0]
            for p in range(1, N_DEV):
                g_amax = jnp.maximum(g_amax, amax_recv[p, 0, 0])
        scale = g_amax / 448.0
        inv_scale = 448.0 / g_amax

        for c, d in enumerate(EP_ORDER):
            o = (my_i + d) % N_DEV
            oslot = c % 2
            if c >= 2:
                o_prev = (my_i + EP_ORDER[c - 2]) % N_DEV
                pltpu.make_async_copy(
                    outv.at[oslot],
                    out_hbm.at[pl.ds(o_prev * m_per, m_per), :],
                    out_sems.at[oslot],
                ).wait()
            if d != 0 and not _NOCOMM:
                for h in range(H):
                    rcv = pltpu.make_async_remote_copy(
                        src_ref=sendbuf.at[0, pl.ds(h * hm, hm), :],
                        dst_ref=recvbuf.at[o, pl.ds(h * hm, hm), :],
                        send_sem=send_sems.at[0],
                        recv_sem=recv_sems.at[o * H + h],
                        device_id=(o,), device_id_type=pl.DeviceIdType.MESH,
                    )
                    rcv.wait_recv()
            v = jnp.maximum(recvbuf[o].astype(jnp.float32), 0.0)
            q = jnp.minimum(v * inv_scale, 448.0).astype(jnp.float8_e4m3fn)
            outv[oslot, :, :] = q.astype(jnp.float32) * scale
            pltpu.make_async_copy(
                outv.at[oslot],
                out_hbm.at[pl.ds(o * m_per, m_per), :],
                out_sems.at[oslot],
            ).start()

        for c in (2, 3):
            o = (my_i + EP_ORDER[c]) % N_DEV
            pltpu.make_async_copy(
                outv.at[c % 2],
                out_hbm.at[pl.ds(o * m_per, m_per), :],
                out_sems.at[c % 2],
            ).wait()
        for sb in range(N_DEV - 1 if not _NOCOMM else 0):
            for h in range(H):
                snd = pltpu.make_async_remote_copy(
                    src_ref=sendbuf.at[sb, pl.ds(h * hm, hm), :],
                    dst_ref=recvbuf.at[0, pl.ds(h * hm, hm), :],
                    send_sem=send_sems.at[sb * H + h],
                    recv_sem=recv_sems.at[0],
                    device_id=(0,), device_id_type=pl.DeviceIdType.MESH,
                )
                snd.wait_send()

    return pl.pallas_call(
        body,
        out_shape=jax.ShapeDtypeStruct((N_DEV * m_per, n_per), jnp.float32),
        in_specs=[
            pl.BlockSpec(memory_space=pl.ANY),
            pl.BlockSpec(memory_space=pl.ANY),
        ],
        out_specs=pl.BlockSpec(memory_space=pl.ANY),
        scratch_shapes=[
            pltpu.VMEM((2, sm, k), jnp.float32),
            pltpu.VMEM((m_per, k), jnp.bfloat16),
            pltpu.VMEM((2, k, n_per), jnp.float32),
            pltpu.VMEM((3, m_per, n_per), jnp.bfloat16),
            pltpu.VMEM((N_DEV, m_per, n_per), jnp.bfloat16),
            pltpu.VMEM((2, m_per, n_per), jnp.float32),
            pltpu.VMEM((8, 128), jnp.float32),
            pltpu.VMEM((N_DEV, 8, 128), jnp.float32),
            pltpu.SemaphoreType.DMA((2,)),
            pltpu.SemaphoreType.DMA((2,)),
            pltpu.SemaphoreType.DMA((2,)),
            pltpu.SemaphoreType.DMA((3 * H,)),
            pltpu.SemaphoreType.DMA((N_DEV * H,)),
            pltpu.SemaphoreType.DMA((N_DEV,)),
            pltpu.SemaphoreType.DMA((N_DEV,)),
        ],
        compiler_params=pltpu.CompilerParams(
            collective_id=None if _NOCOMM else 0,
            vmem_limit_bytes=56 * 1024 * 1024,
        ),
    )(x, w_mat)
